# Initial kernel scaffold; baseline (speedup 1.0000x reference)
import jax
import jax.numpy as jnp
from jax.experimental import pallas as pl


def kernel(x, table):
    raise NotImplementedError("write your pallas kernel here")


# speedup vs baseline: 10.2360x; 10.2360x over previous
import jax
import jax.numpy as jnp
from jax.experimental import pallas as pl

_B, _T = 16, 16000
_N_IN, _N_OUT = 256, 512
_TT = 3200
_NT = _T // _TT


def _onehot_kernel(x_ref, tab_ref, out_ref):
    idx = x_ref[0, 0, 0, :]
    tab = tab_ref[...]
    ss = tab / (1.0 + jnp.abs(tab))
    iota = jax.lax.broadcasted_iota(jnp.int32, (_N_IN, _TT), 0)
    oh = (iota == idx[None, :]).astype(jnp.bfloat16)
    out = jax.lax.dot_general(
        ss.astype(jnp.bfloat16), oh,
        (((0,), (0,)), ((), ())),
        preferred_element_type=jnp.float32,
    )
    out_ref[0, :, :] = out


def kernel(x, table):
    x4 = x.astype(jnp.int32).reshape(_B, _NT, 1, _TT)
    return pl.pallas_call(
        _onehot_kernel,
        grid=(_B, _NT),
        in_specs=[
            pl.BlockSpec((1, 1, 1, _TT), lambda b, t: (b, t, 0, 0)),
            pl.BlockSpec((_N_IN, _N_OUT), lambda b, t: (0, 0)),
        ],
        out_specs=pl.BlockSpec((1, _N_OUT, _TT), lambda b, t: (b, 0, t)),
        out_shape=jax.ShapeDtypeStruct((_B, _N_OUT, _T), jnp.float32),
    )(x4, table)
